# baseline (speedup 1.0000x reference)
import functools

import jax
import jax.numpy as jnp
from jax import lax
from jax.experimental import pallas as pl
from jax.experimental.pallas import tpu as pltpu
from jax.experimental.pallas import tpu_sc as plsc

B, L, TAGS, D = 4096, 50, 20, 16
NC, NS = 2, 16
NW = NC * NS
BW = B // NW
CHUNKS = L
NR = TAGS * BW
KROWS = 4


def _sc_pooled_lookup(xf, table):
    mesh = plsc.VectorSubcoreMesh(core_axis_name="c", subcore_axis_name="s")

    @functools.partial(
        pl.kernel,
        mesh=mesh,
        out_type=jax.ShapeDtypeStruct((L, D // 8, NW, 8, BW), jnp.float32),
        scratch_types=[
            pltpu.VMEM((NR,), jnp.int32),
            pltpu.VMEM((NR,), jnp.int32),
            pltpu.VMEM((NR, D), jnp.float32),
            pltpu.VMEM((NR, D), jnp.float32),
            pltpu.VMEM((D, BW), jnp.float32),
            pltpu.VMEM((D, BW), jnp.float32),
            pltpu.VMEM((BW,), jnp.float32),
            pltpu.VMEM((BW,), jnp.float32),
            pltpu.SemaphoreType.DMA,
            pltpu.SemaphoreType.DMA,
            pltpu.SemaphoreType.DMA,
            pltpu.SemaphoreType.DMA,
            pltpu.SemaphoreType.DMA,
            pltpu.SemaphoreType.DMA,
        ],
        compiler_params=pltpu.CompilerParams(use_tc_tiling_on_sc=False,
                                             needs_layout_passes=False),
    )
    def sc_kernel(xf_hbm, table_hbm, out_hbm, idx_v0, idx_v1, rows_v0,
                  rows_v1, out_v0, out_v1, rv_v0, rv_v1, gsem0, gsem1,
                  isem0, isem1, osem0, osem1):
        idx_vs = (idx_v0, idx_v1)
        rows_vs = (rows_v0, rows_v1)
        out_vs = (out_v0, out_v1)
        rv_vs = (rv_v0, rv_v1)
        gsems = (gsem0, gsem1)
        isems = (isem0, isem1)
        osems = (osem0, osem1)
        wid = lax.axis_index("s") * NC + lax.axis_index("c")
        iota = lax.iota(jnp.int32, 16)

        def fire_idx(ci, b):
            pltpu.async_copy(xf_hbm.at[wid, pl.ds(ci * NR, NR)], idx_vs[b],
                             isems[b])

        def drain_idx(b):
            pltpu.make_async_copy(xf_hbm.at[wid, pl.ds(0, NR)], idx_vs[b],
                                  isems[b]).wait()

        def fire_gath(ci, b):
            for t in range(0, TAGS, KROWS):
                pltpu.async_copy(
                    table_hbm.at[idx_vs[b].at[pl.ds(t * BW, KROWS * BW)]],
                    rows_vs[b].at[pl.ds(t * BW, KROWS * BW)],
                    gsems[b],
                )

        def drain_gath(b):
            pltpu.make_async_copy(
                table_hbm.at[pl.ds(0, NR)], rows_vs[b], gsems[b]
            ).wait()

        def drain_out(b):
            pltpu.make_async_copy(out_hbm.at[0, 0, 0],
                                  out_vs[b].at[pl.ds(0, 8)], osems[b]).wait()
            pltpu.make_async_copy(out_hbm.at[0, 0, 0],
                                  out_vs[b].at[pl.ds(8, 8)], osems[b]).wait()

        def counts(b):
            idx_v = idx_vs[b]
            rv_v = rv_vs[b]

            def cnt_body(q, _):
                b0 = q * 16
                cnt = jnp.zeros((16,), jnp.float32)
                for t in range(TAGS):
                    vals = idx_v[pl.ds(t * BW + b0, 16)]
                    cnt = cnt + (vals > 0).astype(jnp.float32)
                rv_v[pl.ds(b0, 16)] = 1.0 / (cnt + 1e-6)
                return 0

            lax.fori_loop(0, BW // 16, cnt_body, 0)

        def accum(ci, b):
            rows = rows_vs[b]
            rv_v = rv_vs[b]
            out_v = out_vs[b]

            @pl.when(ci >= 2)
            def _():
                drain_out(b)

            def grp_body(q, _):
                b0 = q * 16
                rv = rv_v[pl.ds(b0, 16)]
                for r in range(16):
                    base = b0 + r
                    acc = rows[base]
                    for t in range(1, TAGS):
                        acc = acc + rows[t * BW + base]
                    plsc.store_scatter(
                        out_v, [iota, jnp.full((16,), base, jnp.int32)],
                        acc * rv[r])
                return 0

            lax.fori_loop(0, BW // 16, grp_body, 0)
            pltpu.async_copy(out_v.at[pl.ds(0, 8)], out_hbm.at[ci, 0, wid],
                             osems[b])
            pltpu.async_copy(out_v.at[pl.ds(8, 8)], out_hbm.at[ci, 1, wid],
                             osems[b])

        fire_idx(0, 0)
        drain_idx(0)
        fire_gath(0, 0)
        fire_idx(1, 1)

        def pair_body(p, _):
            ca = 2 * p
            cb = ca + 1
            drain_idx(1)
            drain_gath(0)
            fire_gath(cb, 1)
            counts(0)

            @pl.when(ca + 2 < CHUNKS)
            def _():
                fire_idx(ca + 2, 0)

            accum(ca, 0)
            @pl.when(ca + 2 < CHUNKS)
            def _():
                drain_idx(0)

            drain_gath(1)

            @pl.when(ca + 2 < CHUNKS)
            def _():
                fire_gath(ca + 2, 0)

            counts(1)

            @pl.when(cb + 2 < CHUNKS)
            def _():
                fire_idx(cb + 2, 1)

            accum(cb, 1)
            return 0

        lax.fori_loop(0, CHUNKS // 2, pair_body, 0)
        drain_out(0)
        drain_out(1)

    return sc_kernel(xf, table)


def kernel(x, table):
    xf = (x.reshape(NW, BW, L, TAGS)
           .transpose(0, 2, 3, 1)
           .reshape(NW, L * TAGS * BW))
    out5 = _sc_pooled_lookup(xf, table)
    return out5.transpose(2, 4, 0, 1, 3).reshape(B, L, D)


# speedup vs baseline: 1.0010x; 1.0010x over previous
import functools

import jax
import jax.numpy as jnp
from jax import lax
from jax.experimental import pallas as pl
from jax.experimental.pallas import tpu as pltpu
from jax.experimental.pallas import tpu_sc as plsc

B, L, TAGS, D = 4096, 50, 20, 16
NC, NS = 2, 16
NW = NC * NS
BW = B // NW
CHUNKS = L
NR = TAGS * BW
KROWS = 10


def _sc_pooled_lookup(xf, table):
    mesh = plsc.VectorSubcoreMesh(core_axis_name="c", subcore_axis_name="s")

    @functools.partial(
        pl.kernel,
        mesh=mesh,
        out_type=jax.ShapeDtypeStruct((L, D // 8, NW, 8, BW), jnp.float32),
        scratch_types=[
            pltpu.VMEM((NR,), jnp.int32),
            pltpu.VMEM((NR,), jnp.int32),
            pltpu.VMEM((NR, D), jnp.float32),
            pltpu.VMEM((NR, D), jnp.float32),
            pltpu.VMEM((D, BW), jnp.float32),
            pltpu.VMEM((D, BW), jnp.float32),
            pltpu.VMEM((BW,), jnp.float32),
            pltpu.VMEM((BW,), jnp.float32),
            pltpu.SemaphoreType.DMA,
            pltpu.SemaphoreType.DMA,
            pltpu.SemaphoreType.DMA,
            pltpu.SemaphoreType.DMA,
            pltpu.SemaphoreType.DMA,
            pltpu.SemaphoreType.DMA,
        ],
        compiler_params=pltpu.CompilerParams(use_tc_tiling_on_sc=False,
                                             needs_layout_passes=False),
    )
    def sc_kernel(xf_hbm, table_hbm, out_hbm, idx_v0, idx_v1, rows_v0,
                  rows_v1, out_v0, out_v1, rv_v0, rv_v1, gsem0, gsem1,
                  isem0, isem1, osem0, osem1):
        idx_vs = (idx_v0, idx_v1)
        rows_vs = (rows_v0, rows_v1)
        out_vs = (out_v0, out_v1)
        rv_vs = (rv_v0, rv_v1)
        gsems = (gsem0, gsem1)
        isems = (isem0, isem1)
        osems = (osem0, osem1)
        wid = lax.axis_index("s") * NC + lax.axis_index("c")
        iota = lax.iota(jnp.int32, 16)

        def fire_idx(ci, b):
            pltpu.async_copy(xf_hbm.at[wid, pl.ds(ci * NR, NR)], idx_vs[b],
                             isems[b])

        def drain_idx(b):
            pltpu.make_async_copy(xf_hbm.at[wid, pl.ds(0, NR)], idx_vs[b],
                                  isems[b]).wait()

        def fire_gath(ci, b):
            for t in range(0, TAGS, KROWS):
                pltpu.async_copy(
                    table_hbm.at[idx_vs[b].at[pl.ds(t * BW, KROWS * BW)]],
                    rows_vs[b].at[pl.ds(t * BW, KROWS * BW)],
                    gsems[b],
                )

        def drain_gath(b):
            pltpu.make_async_copy(
                table_hbm.at[pl.ds(0, NR)], rows_vs[b], gsems[b]
            ).wait()

        def drain_out(b):
            pltpu.make_async_copy(out_hbm.at[0, 0, 0],
                                  out_vs[b].at[pl.ds(0, 8)], osems[b]).wait()
            pltpu.make_async_copy(out_hbm.at[0, 0, 0],
                                  out_vs[b].at[pl.ds(8, 8)], osems[b]).wait()

        def counts(b):
            idx_v = idx_vs[b]
            rv_v = rv_vs[b]

            def cnt_body(q, _):
                b0 = q * 16
                cnt = jnp.zeros((16,), jnp.float32)
                for t in range(TAGS):
                    vals = idx_v[pl.ds(t * BW + b0, 16)]
                    cnt = cnt + (vals > 0).astype(jnp.float32)
                rv_v[pl.ds(b0, 16)] = 1.0 / (cnt + 1e-6)
                return 0

            lax.fori_loop(0, BW // 16, cnt_body, 0)

        def accum(ci, b):
            rows = rows_vs[b]
            rv_v = rv_vs[b]
            out_v = out_vs[b]

            @pl.when(ci >= 2)
            def _():
                drain_out(b)

            def grp_body(q, _):
                b0 = q * 16
                rv = rv_v[pl.ds(b0, 16)]
                for r in range(16):
                    base = b0 + r
                    acc = rows[base]
                    for t in range(1, TAGS):
                        acc = acc + rows[t * BW + base]
                    plsc.store_scatter(
                        out_v, [iota, jnp.full((16,), base, jnp.int32)],
                        acc * rv[r])
                return 0

            lax.fori_loop(0, BW // 16, grp_body, 0)
            pltpu.async_copy(out_v.at[pl.ds(0, 8)], out_hbm.at[ci, 0, wid],
                             osems[b])
            pltpu.async_copy(out_v.at[pl.ds(8, 8)], out_hbm.at[ci, 1, wid],
                             osems[b])

        fire_idx(0, 0)
        drain_idx(0)
        fire_gath(0, 0)
        fire_idx(1, 1)

        def pair_body(p, _):
            ca = 2 * p
            cb = ca + 1
            drain_idx(1)
            drain_gath(0)
            fire_gath(cb, 1)
            counts(0)

            @pl.when(ca + 2 < CHUNKS)
            def _():
                fire_idx(ca + 2, 0)

            accum(ca, 0)
            @pl.when(ca + 2 < CHUNKS)
            def _():
                drain_idx(0)

            drain_gath(1)

            @pl.when(ca + 2 < CHUNKS)
            def _():
                fire_gath(ca + 2, 0)

            counts(1)

            @pl.when(cb + 2 < CHUNKS)
            def _():
                fire_idx(cb + 2, 1)

            accum(cb, 1)
            return 0

        lax.fori_loop(0, CHUNKS // 2, pair_body, 0)
        drain_out(0)
        drain_out(1)

    return sc_kernel(xf, table)


def kernel(x, table):
    xf = (x.reshape(NW, BW, L, TAGS)
           .transpose(0, 2, 3, 1)
           .reshape(NW, L * TAGS * BW))
    out5 = _sc_pooled_lookup(xf, table)
    return out5.transpose(2, 4, 0, 1, 3).reshape(B, L, D)
